# baseline (speedup 1.0000x reference)
import jax
import jax.numpy as jnp
from jax import lax
from jax.experimental import pallas as pl
from jax.experimental.pallas import tpu as pltpu
from jax.experimental.pallas import tpu_sc as plsc

NC = 2
NS = 16
L = 16
NW = NC * NS
BATCH = 16384
BPW = BATCH // NW
D = 32
NROWS = 1000000
CHUNK = 128
NCHUNK = BPW // CHUNK
GPC = CHUNK // L


def _body(x_hbm, w_hbm, h_hbm, out_hbm,
          xv, iu, iv, ub, vb, res, sem_u, sem_v):
    cid = lax.axis_index("c")
    sid = lax.axis_index("s")
    wid = sid * NC + cid
    base = wid * BPW

    pltpu.sync_copy(x_hbm.at[pl.ds(base, BPW)], xv.at[0])
    pltpu.sync_copy(x_hbm.at[pl.ds(BATCH + base, BPW)], xv.at[1])

    def fire(k, s):
        def gen16(j, _):
            i0 = k * CHUNK + j * L
            a = xv[0, pl.ds(i0, L)]
            b = xv[1, pl.ds(i0, L)]

            def percol(c, ab):
                a_c, b_c = ab
                iu[s, c, pl.ds(j * L, L)] = a_c
                iv[s, c, pl.ds(j * L, L)] = b_c
                return a_c + NROWS, b_c + NROWS

            lax.fori_loop(0, D, percol, (a, b), unroll=4)
            return 0

        lax.fori_loop(0, CHUNK // L, gen16, 0)

        def enq(c, _):
            pltpu.async_copy(w_hbm.at[iu.at[s, c]], ub.at[s, c], sem_u)
            pltpu.async_copy(h_hbm.at[iv.at[s, c]], vb.at[s, c], sem_v)
            return 0

        lax.fori_loop(0, D, enq, 0)

    def drain(s):
        pltpu.make_async_copy(w_hbm.at[pl.ds(0, D * CHUNK)],
                              ub.at[s], sem_u).wait()
        pltpu.make_async_copy(h_hbm.at[pl.ds(0, D * CHUNK)],
                              vb.at[s], sem_v).wait()

    def compute(k, s):
        def group(g, _):
            def col(c, acc):
                u = ub[s, c, pl.ds(g * L, L)]
                v = vb[s, c, pl.ds(g * L, L)]
                return acc + u * v

            acc = lax.fori_loop(0, D, col, jnp.zeros((L,), jnp.float32),
                                unroll=4)
            res[pl.ds((k * GPC + g) * L, L)] = 1.0 / (1.0 + jnp.exp(-acc))
            return 0

        lax.fori_loop(0, GPC, group, 0)

    fire(0, 0)
    for k in range(NCHUNK):
        s = k % 2
        drain(s)
        if k + 1 < NCHUNK:
            fire(k + 1, (k + 1) % 2)
        compute(k, s)

    pltpu.sync_copy(res, out_hbm.at[pl.ds(base, BPW)])


@jax.jit
def kernel(x, W, H):
    mesh = plsc.VectorSubcoreMesh(
        core_axis_name="c", subcore_axis_name="s",
        num_cores=NC, num_subcores=NS)
    run = pl.kernel(
        _body,
        out_type=jax.ShapeDtypeStruct((BATCH,), jnp.float32),
        mesh=mesh,
        compiler_params=pltpu.CompilerParams(
            needs_layout_passes=False, use_tc_tiling_on_sc=False),
        scratch_types=[
            pltpu.VMEM((2, BPW), jnp.int32),
            pltpu.VMEM((2, D, CHUNK), jnp.int32),
            pltpu.VMEM((2, D, CHUNK), jnp.int32),
            pltpu.VMEM((2, D, CHUNK), jnp.float32),
            pltpu.VMEM((2, D, CHUNK), jnp.float32),
            pltpu.VMEM((BPW,), jnp.float32),
            pltpu.SemaphoreType.DMA,
            pltpu.SemaphoreType.DMA,
        ],
    )
    return run(x.T.reshape(-1), W.T.reshape(-1), H.T.reshape(-1))


# speedup vs baseline: 5.6763x; 5.6763x over previous
import jax
import jax.numpy as jnp
from jax import lax
from jax.experimental import pallas as pl
from jax.experimental import layout as jex_layout
from jax.experimental.pallas import tpu as pltpu
from jax.experimental.pallas import tpu_sc as plsc

NC = 2
NS = 16
L = 16
NW = NC * NS
BATCH = 16384
BPW = BATCH // NW
D = 32
WIDE = 128
CHUNK = 128
NCHUNK = BPW // CHUNK
GPC = CHUNK // L


def _body(x_hbm, w_hbm, h_hbm, out_hbm,
          xv, uq, vq, useg, vseg, ub, vb, res, sem_u, sem_v):
    cid = lax.axis_index("c")
    sid = lax.axis_index("s")
    wid = sid * NC + cid
    base = wid * BPW

    pltpu.sync_copy(x_hbm.at[pl.ds(base * 2, BPW * 2)], xv)

    lanes = lax.iota(jnp.int32, L)

    def deint(j, _):
        pos = jnp.full((L,), 2 * j * L, jnp.int32) + 2 * lanes
        u = plsc.load_gather(xv, [pos])
        v = plsc.load_gather(xv, [pos + 1])
        k = j // (CHUNK // L)
        o = (j % (CHUNK // L)) * L
        uq[k, pl.ds(o, L)] = u >> 2
        vq[k, pl.ds(o, L)] = v >> 2
        useg[pl.ds(j * L, L)] = (u & 3) * D
        vseg[pl.ds(j * L, L)] = (v & 3) * D
        return 0

    lax.fori_loop(0, BPW // L, deint, 0, unroll=2)

    def fire(k, s):
        cu = pltpu.async_copy(w_hbm.at[uq.at[k]], ub.at[s], sem_u)
        cv = pltpu.async_copy(h_hbm.at[vq.at[k]], vb.at[s], sem_v)
        return cu, cv

    def drain(cu, cv):
        cu.wait()
        cv.wait()

    def compute(k, s):
        def group(g, _):
            gg = k * GPC + g
            row = jnp.full((L,), g * L, jnp.int32) + lanes
            us = useg[pl.ds(gg * L, L)]
            vs = vseg[pl.ds(gg * L, L)]

            def col(c, acc):
                cv = (jnp.full((L,), c, jnp.int32) + lanes) & (D - 1)
                u = plsc.load_gather(ub.at[s], [row, us + cv])
                v = plsc.load_gather(vb.at[s], [row, vs + cv])
                return acc + u * v

            acc = lax.fori_loop(0, D, col, jnp.zeros((L,), jnp.float32),
                                unroll=4)
            res[pl.ds(gg * L, L)] = 1.0 / (1.0 + jnp.exp(-acc))
            return 0

        lax.fori_loop(0, GPC, group, 0)

    inflight = fire(0, 0)
    for k in range(NCHUNK):
        s = k % 2
        drain(*inflight)
        if k + 1 < NCHUNK:
            inflight = fire(k + 1, (k + 1) % 2)
        compute(k, s)

    pltpu.sync_copy(res, out_hbm.at[pl.ds(base, BPW)])


@jax.jit
def kernel(x, W, H):
    mesh = plsc.VectorSubcoreMesh(
        core_axis_name="c", subcore_axis_name="s",
        num_cores=NC, num_subcores=NS)
    run = pl.kernel(
        _body,
        out_type=jax.ShapeDtypeStruct((BATCH,), jnp.float32),
        mesh=mesh,
        compiler_params=pltpu.CompilerParams(
            needs_layout_passes=False, use_tc_tiling_on_sc=True),
        scratch_types=[
            pltpu.VMEM((BPW * 2,), jnp.int32),
            pltpu.VMEM((NCHUNK, CHUNK), jnp.int32),
            pltpu.VMEM((NCHUNK, CHUNK), jnp.int32),
            pltpu.VMEM((BPW,), jnp.int32),
            pltpu.VMEM((BPW,), jnp.int32),
            pltpu.VMEM((2, CHUNK, WIDE), jnp.float32),
            pltpu.VMEM((2, CHUNK, WIDE), jnp.float32),
            pltpu.VMEM((BPW,), jnp.float32),
            pltpu.SemaphoreType.DMA,
            pltpu.SemaphoreType.DMA,
        ],
    )
    lay = jex_layout.Layout(major_to_minor=(0, 1), tiling=((8, 128),))
    W2, H2 = jex_layout.with_layout_constraint(
        (W.reshape(-1, WIDE), H.reshape(-1, WIDE)), (lay, lay))
    return run(x.reshape(-1), W2, H2)


# speedup vs baseline: 8.4341x; 1.4858x over previous
import jax
import jax.numpy as jnp
from jax import lax
from jax.experimental import pallas as pl
from jax.experimental.pallas import tpu as pltpu
from jax.experimental.pallas import tpu_sc as plsc

NC = 2
NS = 16
L = 16
NW = NC * NS
BATCH = 16384
BPW = BATCH // NW
D = 32
CHUNK = 128
NCHUNK = BPW // CHUNK
GPC = CHUNK // L


def _body(x_hbm, w_hbm, h_hbm, out_hbm,
          xv, ub, vb, res, sem_u, sem_v):
    cid = lax.axis_index("c")
    sid = lax.axis_index("s")
    wid = sid * NC + cid
    base = wid * BPW

    pltpu.sync_copy(x_hbm.at[pl.ds(base * 2, BPW * 2)], xv)

    def fire(k, s):
        def issue16(j, _):
            i0 = k * CHUNK + j * L
            a = xv[pl.ds(2 * i0, L)]
            b = xv[pl.ds(2 * i0 + L, L)]
            for t in range(L // 2):
                r = j * L + t
                pltpu.async_copy(w_hbm.at[pl.ds(a[2 * t], 1)],
                                 ub.at[s, pl.ds(r, 1)], sem_u)
                pltpu.async_copy(h_hbm.at[pl.ds(a[2 * t + 1], 1)],
                                 vb.at[s, pl.ds(r, 1)], sem_v)
            for t in range(L // 2):
                r = j * L + L // 2 + t
                pltpu.async_copy(w_hbm.at[pl.ds(b[2 * t], 1)],
                                 ub.at[s, pl.ds(r, 1)], sem_u)
                pltpu.async_copy(h_hbm.at[pl.ds(b[2 * t + 1], 1)],
                                 vb.at[s, pl.ds(r, 1)], sem_v)
            return 0

        lax.fori_loop(0, CHUNK // L, issue16, 0)

    def drain(s):
        pltpu.make_async_copy(w_hbm.at[pl.ds(0, CHUNK)], ub.at[s],
                              sem_u).wait()
        pltpu.make_async_copy(h_hbm.at[pl.ds(0, CHUNK)], vb.at[s],
                              sem_v).wait()

    lanes = lax.iota(jnp.int32, L)

    def compute(k, s):
        def group(g, _):
            row = jnp.full((L,), g * L, jnp.int32) + lanes

            def col(c, acc):
                cv = (jnp.full((L,), c, jnp.int32) + lanes) & (D - 1)
                u = plsc.load_gather(ub.at[s], [row, cv])
                v = plsc.load_gather(vb.at[s], [row, cv])
                return acc + u * v

            acc = lax.fori_loop(0, D, col, jnp.zeros((L,), jnp.float32),
                                unroll=4)
            res[pl.ds((k * GPC + g) * L, L)] = 1.0 / (1.0 + jnp.exp(-acc))
            return 0

        lax.fori_loop(0, GPC, group, 0)

    fire(0, 0)
    for k in range(NCHUNK):
        s = k % 2
        drain(s)
        if k + 1 < NCHUNK:
            fire(k + 1, (k + 1) % 2)
        compute(k, s)

    pltpu.sync_copy(res, out_hbm.at[pl.ds(base, BPW)])


@jax.jit
def kernel(x, W, H):
    mesh = plsc.VectorSubcoreMesh(
        core_axis_name="c", subcore_axis_name="s",
        num_cores=NC, num_subcores=NS)
    run = pl.kernel(
        _body,
        out_type=jax.ShapeDtypeStruct((BATCH,), jnp.float32),
        mesh=mesh,
        compiler_params=pltpu.CompilerParams(
            needs_layout_passes=False, use_tc_tiling_on_sc=True),
        scratch_types=[
            pltpu.VMEM((BPW * 2,), jnp.int32),
            pltpu.VMEM((2, CHUNK, D), jnp.float32),
            pltpu.VMEM((2, CHUNK, D), jnp.float32),
            pltpu.VMEM((BPW,), jnp.float32),
            pltpu.SemaphoreType.DMA,
            pltpu.SemaphoreType.DMA,
        ],
    )
    return run(x.reshape(-1), W, H)
